# baseline (speedup 1.0000x reference)
import functools
import jax
import jax.numpy as jnp
from jax import lax
from jax.experimental import pallas as pl
from jax.experimental.pallas import tpu as pltpu
from jax.experimental.pallas import tpu_sc as plsc

B = 8
M = 64
G = 64
F = 192
R = B * M

NC = 2
NW = 32

_DOT_RT = (((1,), (1,)), ((), ()))
_DOT_LT = (((0,), (0,)), ((), ()))


def _sc_argmin(pts_hbm, gsplat):
    mesh = plsc.VectorSubcoreMesh(core_axis_name="c", subcore_axis_name="s")

    @functools.partial(
        pl.kernel, mesh=mesh,
        out_type=jax.ShapeDtypeStruct((R,), jnp.int32),
        scratch_types=[
            pltpu.VMEM((3, G, 16), jnp.float32),
            pltpu.VMEM((3, 16), jnp.float32),
            pltpu.VMEM((16,), jnp.int32),
        ],
    )
    def k(pts_ref, gs_ref, out_ref, gs_v, xyz_v, idx_v):
        wid = lax.axis_index("s") * NC + lax.axis_index("c")
        b = wid // 4
        ch = wid % 4
        pltpu.sync_copy(gs_ref, gs_v)
        for k3 in range(3):
            pltpu.sync_copy(pts_ref.at[k3, b, pl.ds(ch * 16, 16)],
                            xyz_v.at[k3])
        x = xyz_v[0]
        y = xyz_v[1]
        z = xyz_v[2]
        best = jnp.full((16,), 3.4e38, jnp.float32)
        bidx = jnp.zeros((16,), jnp.int32)
        for g in range(G):
            dx = x - gs_v[0, g]
            dy = y - gs_v[1, g]
            dz = z - gs_v[2, g]
            dist = dx * dx + dy * dy + dz * dz
            m = dist < best
            best = jnp.where(m, dist, best)
            bidx = jnp.where(m, jnp.full((16,), g, jnp.int32), bidx)
        idx_v[...] = bidx
        pltpu.sync_copy(idx_v, out_ref.at[pl.ds(wid * 16, 16)])

    return k(pts_hbm, gsplat)


def _qc_kernel(pc_ref, nm_ref, co_ref, idx_ref,
               ptw1_ref, ptb1_ref, ptw2_ref, ptb2_ref, ptw3_ref, ptb3_ref,
               nmw1_ref, nmb1_ref, nmw2_ref, nmb2_ref, nmw3_ref, nmb3_ref,
               txw1_ref, txb1_ref, txw2_ref, txb2_ref, txw3_ref, txb3_ref,
               dnw1t_ref, dnb1_ref, dnw2_ref, dnb2_ref,
               clw1t_ref, clb1_ref, clw2t_ref, clb2_ref,
               svw1t_ref, svb1_ref, svw2t_ref, svb2_ref,
               probs_ref, sev_ref, proc_ref, gf_ref):
    def coords(ref):
        x24 = ref[...].reshape(3 * B, 128)
        rows = [
            jnp.concatenate([x24[k * B + b:k * B + b + 1, :M]
                             for b in range(B)], axis=1)
            for k in range(3)
        ]
        return jnp.concatenate(rows, axis=0)

    pts3 = coords(pc_ref)
    nrm3 = coords(nm_ref)
    col3 = coords(co_ref)

    def rowvec(ref):
        return ref[...].reshape(1, -1)

    def matT(x, wt_ref):
        return jax.lax.dot_general(x, wt_ref[...], _DOT_RT,
                                   preferred_element_type=jnp.float32)

    def mlp(x3, w1, b1, w2, b2, w3, b3):
        h = jax.lax.dot_general(x3, w1[...], _DOT_LT,
                                preferred_element_type=jnp.float32)
        h = jnp.maximum(h + rowvec(b1), 0.0)
        h = jnp.maximum(h @ w2[...] + rowvec(b2), 0.0)
        return h @ w3[...] + rowvec(b3)

    pf = mlp(pts3, ptw1_ref, ptb1_ref, ptw2_ref, ptb2_ref, ptw3_ref, ptb3_ref)
    nf = mlp(nrm3, nmw1_ref, nmb1_ref, nmw2_ref, nmb2_ref, nmw3_ref, nmb3_ref)
    tf = mlp(col3, txw1_ref, txb1_ref, txw2_ref, txb2_ref, txw3_ref, txb3_ref)
    comb = jnp.concatenate([pf, nf, tf], axis=1)

    idx = idx_ref[...].reshape(1, R)

    cell_col = jax.lax.broadcasted_iota(jnp.int32, (G, M), 0)
    lane_io = jax.lax.broadcasted_iota(jnp.int32, (G, M), 1)

    gfs = []
    for b in range(B):
        idx_b = idx[:, b * M:(b + 1) * M]
        onehot = cell_col == idx_b
        val = jnp.where(onehot, lane_io + 1, 0)
        wins = jnp.max(val, axis=1, keepdims=True)
        sel = ((val == wins) & (wins > 0)).astype(jnp.float32)
        gf_b = sel @ comb[b * M:(b + 1) * M]
        gf_ref[b, :, :] = gf_b
        gfs.append(gf_b)
    gf = jnp.concatenate(gfs, axis=0)

    hd = jnp.maximum(matT(gf, dnw1t_ref) + rowvec(dnb1_ref), 0.0)
    defect = hd @ dnw2_ref[...] + rowvec(dnb2_ref)

    proc_ref[...] = jnp.transpose(defect.reshape(B, G, 64), (0, 2, 1))

    hc = jnp.maximum(matT(defect, clw1t_ref) + rowvec(clb1_ref), 0.0)
    logits = matT(hc, clw2t_ref) + rowvec(clb2_ref)
    probs = jax.nn.softmax(logits, axis=-1)
    probs3 = jnp.transpose(probs.reshape(B, M, 5), (0, 2, 1))
    for b in range(B):
        probs_ref[:, b, :] = probs3[b]

    hs = jnp.maximum(matT(defect, svw1t_ref) + rowvec(svb1_ref), 0.0)
    sev_pre = jnp.sum(hs * rowvec(svw2t_ref), axis=1, keepdims=True)
    sev = jax.nn.sigmoid(sev_pre + svb2_ref[...])
    sev3 = jnp.transpose(sev.reshape(B, M, 1), (0, 2, 1))
    for b in range(B):
        sev_ref[b:b + 1, :] = sev3[b]


def kernel(point_cloud, normals, colors, grid_points, params):
    t = jnp.transpose
    pts_t = t(point_cloud, (2, 0, 1))
    nrm_t = t(normals, (2, 0, 1))
    col_t = t(colors, (2, 0, 1))

    gsplat = jnp.broadcast_to(t(grid_points).reshape(3, G, 1), (3, G, 16))
    idx = _sc_argmin(pts_t, gsplat)

    p = params
    weight_args = [
        p["pt_W1"], p["pt_b1"], p["pt_W2"], p["pt_b2"], p["pt_W3"], p["pt_b3"],
        p["nm_W1"], p["nm_b1"], p["nm_W2"], p["nm_b2"], p["nm_W3"], p["nm_b3"],
        p["tx_W1"], p["tx_b1"], p["tx_W2"], p["tx_b2"], p["tx_W3"], p["tx_b3"],
        t(p["dn_W1"]), p["dn_b1"], p["dn_W2"], p["dn_b2"],
        t(p["cl_W1"]), p["cl_b1"], t(p["cl_W2"]), p["cl_b2"],
        t(p["sv_W1"]), p["sv_b1"], t(p["sv_W2"]), p["sv_b2"].reshape(1, 1),
    ]

    out_shapes = (
        jax.ShapeDtypeStruct((5, B, G), jnp.float32),
        jax.ShapeDtypeStruct((B, G), jnp.float32),
        jax.ShapeDtypeStruct((B, 64, G), jnp.float32),
        jax.ShapeDtypeStruct((B, G, F), jnp.float32),
    )

    first64 = pl.BlockSpec((3, B, 128), lambda i: (0, 0, 0))
    full = lambda a: pl.BlockSpec(a.shape, lambda i: (0,) * a.ndim)

    probs5, sev, proc, gf = pl.pallas_call(
        _qc_kernel,
        out_shape=out_shapes,
        grid=(1,),
        in_specs=[first64, first64, first64, full(idx)]
        + [full(w) for w in weight_args],
        out_specs=tuple(
            pl.BlockSpec(s.shape, lambda i, n=len(s.shape): (0,) * n)
            for s in out_shapes),
    )(pts_t, nrm_t, col_t, idx, *weight_args)

    return (jnp.transpose(probs5, (1, 2, 0)), sev, proc,
            jnp.transpose(gf, (0, 2, 1)))


# speedup vs baseline: 5.6783x; 5.6783x over previous
import jax
import jax.numpy as jnp
from jax.experimental import pallas as pl

B = 8
M = 64
G = 64
F = 192
R = B * M

_DOT_RT = (((1,), (1,)), ((), ()))
_DOT_LT = (((0,), (0,)), ((), ()))


def _qc_kernel(pc_ref, nm_ref, co_ref, gpt_ref,
               ptw1_ref, ptb1_ref, ptw2_ref, ptb2_ref, ptw3_ref, ptb3_ref,
               nmw1_ref, nmb1_ref, nmw2_ref, nmb2_ref, nmw3_ref, nmb3_ref,
               txw1_ref, txb1_ref, txw2_ref, txb2_ref, txw3_ref, txb3_ref,
               dnw1t_ref, dnb1_ref, dnw2_ref, dnb2_ref,
               clw1t_ref, clb1_ref, clw2t_ref, clb2_ref,
               svw1t_ref, svb1_ref, svw2t_ref, svb2_ref,
               probs_ref, sev_ref, proc_ref, gf_ref):
    def coords(ref):
        x24 = ref[...].reshape(3 * B, 128)
        rows = [
            jnp.concatenate([x24[k * B + b:k * B + b + 1, :M]
                             for b in range(B)], axis=1)
            for k in range(3)
        ]
        return jnp.concatenate(rows, axis=0)

    pts3 = coords(pc_ref)
    nrm3 = coords(nm_ref)
    col3 = coords(co_ref)
    gp = jnp.transpose(gpt_ref[...])

    def rowvec(ref):
        return ref[...].reshape(1, -1)

    def matT(x, wt_ref):
        return jax.lax.dot_general(x, wt_ref[...], _DOT_RT,
                                   preferred_element_type=jnp.float32)

    def mlp(x3, w1, b1, w2, b2, w3, b3):
        h = jax.lax.dot_general(x3, w1[...], _DOT_LT,
                                preferred_element_type=jnp.float32)
        h = jnp.maximum(h + rowvec(b1), 0.0)
        h = jnp.maximum(h @ w2[...] + rowvec(b2), 0.0)
        return h @ w3[...] + rowvec(b3)

    pf = mlp(pts3, ptw1_ref, ptb1_ref, ptw2_ref, ptb2_ref, ptw3_ref, ptb3_ref)
    nf = mlp(nrm3, nmw1_ref, nmb1_ref, nmw2_ref, nmb2_ref, nmw3_ref, nmb3_ref)
    tf = mlp(col3, txw1_ref, txb1_ref, txw2_ref, txb2_ref, txw3_ref, txb3_ref)
    comb = jnp.concatenate([pf, nf, tf], axis=1)

    dT = ((gp[:, 0:1] - pts3[0:1, :]) ** 2
          + (gp[:, 1:2] - pts3[1:2, :]) ** 2
          + (gp[:, 2:3] - pts3[2:3, :]) ** 2)
    minv = jnp.min(dT, axis=0, keepdims=True)
    gio_s = jax.lax.broadcasted_iota(jnp.int32, (G, R), 0)
    idx = jnp.min(jnp.where(dT == minv, gio_s, G), axis=0, keepdims=True)

    cell_col = jax.lax.broadcasted_iota(jnp.int32, (G, M), 0)
    lane_io = jax.lax.broadcasted_iota(jnp.int32, (G, M), 1)

    gfs = []
    for b in range(B):
        idx_b = idx[:, b * M:(b + 1) * M]
        onehot = cell_col == idx_b
        val = jnp.where(onehot, lane_io + 1, 0)
        wins = jnp.max(val, axis=1, keepdims=True)
        sel = ((val == wins) & (wins > 0)).astype(jnp.float32)
        gf_b = sel @ comb[b * M:(b + 1) * M]
        gf_ref[b, :, :] = gf_b
        gfs.append(gf_b)
    gf = jnp.concatenate(gfs, axis=0)

    hd = jnp.maximum(matT(gf, dnw1t_ref) + rowvec(dnb1_ref), 0.0)
    defect = hd @ dnw2_ref[...] + rowvec(dnb2_ref)

    proc_ref[...] = jnp.transpose(defect.reshape(B, G, 64), (0, 2, 1))

    hc = jnp.maximum(matT(defect, clw1t_ref) + rowvec(clb1_ref), 0.0)
    logits = matT(hc, clw2t_ref) + rowvec(clb2_ref)
    probs = jax.nn.softmax(logits, axis=-1)
    probs3 = jnp.transpose(probs.reshape(B, M, 5), (0, 2, 1))
    for b in range(B):
        probs_ref[:, b, :] = probs3[b]

    hs = jnp.maximum(matT(defect, svw1t_ref) + rowvec(svb1_ref), 0.0)
    sev_pre = jnp.sum(hs * rowvec(svw2t_ref), axis=1, keepdims=True)
    sev = jax.nn.sigmoid(sev_pre + svb2_ref[...])
    sev3 = jnp.transpose(sev.reshape(B, M, 1), (0, 2, 1))
    for b in range(B):
        sev_ref[b:b + 1, :] = sev3[b]


def kernel(point_cloud, normals, colors, grid_points, params):
    t = jnp.transpose
    pts_t = t(point_cloud, (2, 0, 1))
    nrm_t = t(normals, (2, 0, 1))
    col_t = t(colors, (2, 0, 1))

    p = params
    weight_args = [
        p["pt_W1"], p["pt_b1"], p["pt_W2"], p["pt_b2"], p["pt_W3"], p["pt_b3"],
        p["nm_W1"], p["nm_b1"], p["nm_W2"], p["nm_b2"], p["nm_W3"], p["nm_b3"],
        p["tx_W1"], p["tx_b1"], p["tx_W2"], p["tx_b2"], p["tx_W3"], p["tx_b3"],
        t(p["dn_W1"]), p["dn_b1"], p["dn_W2"], p["dn_b2"],
        t(p["cl_W1"]), p["cl_b1"], t(p["cl_W2"]), p["cl_b2"],
        t(p["sv_W1"]), p["sv_b1"], t(p["sv_W2"]), p["sv_b2"].reshape(1, 1),
    ]

    out_shapes = (
        jax.ShapeDtypeStruct((5, B, G), jnp.float32),
        jax.ShapeDtypeStruct((B, G), jnp.float32),
        jax.ShapeDtypeStruct((B, 64, G), jnp.float32),
        jax.ShapeDtypeStruct((B, G, F), jnp.float32),
    )

    first64 = pl.BlockSpec((3, B, 128), lambda i: (0, 0, 0))
    full = lambda a: pl.BlockSpec(a.shape, lambda i: (0,) * a.ndim)
    gpt = t(grid_points)

    probs5, sev, proc, gf = pl.pallas_call(
        _qc_kernel,
        out_shape=out_shapes,
        grid=(1,),
        in_specs=[first64, first64, first64, full(gpt)]
        + [full(w) for w in weight_args],
        out_specs=tuple(
            pl.BlockSpec(s.shape, lambda i, n=len(s.shape): (0,) * n)
            for s in out_shapes),
    )(pts_t, nrm_t, col_t, gpt, *weight_args)

    return (jnp.transpose(probs5, (1, 2, 0)), sev, proc,
            jnp.transpose(gf, (0, 2, 1)))
